# baseline (speedup 1.0000x reference)
import jax
import jax.numpy as jnp
from jax.experimental import pallas as pl


def _copy_body(e_ref, e_out_ref):
    e_out_ref[...] = e_ref[...]


def kernel(x, edge_index, edge_attr):
    del edge_index
    n_edges, d_edge = edge_attr.shape
    grid = 25
    be = n_edges // grid
    e_out = pl.pallas_call(
        _copy_body,
        grid=(grid,),
        out_shape=jax.ShapeDtypeStruct(edge_attr.shape, edge_attr.dtype),
        in_specs=[pl.BlockSpec((be, d_edge), lambda i: (i, 0))],
        out_specs=pl.BlockSpec((be, d_edge), lambda i: (i, 0)),
    )(edge_attr)
    return (x, e_out)


# speedup vs baseline: 10.2967x; 10.2967x over previous
import jax
import jax.numpy as jnp
from jax.experimental import pallas as pl


def _copy_body(x_ref, e_ref, x_out_ref, e_out_ref):
    x_out_ref[...] = x_ref[...]
    e_out_ref[...] = e_ref[...]


def kernel(x, edge_index, edge_attr):
    del edge_index
    n_nodes, d_feat = x.shape
    n_edges, d_edge = edge_attr.shape
    e_t = edge_attr.T

    grid = 25
    bx = n_nodes // grid
    be = n_edges // grid

    x_out, e_out_t = pl.pallas_call(
        _copy_body,
        grid=(grid,),
        out_shape=(
            jax.ShapeDtypeStruct(x.shape, x.dtype),
            jax.ShapeDtypeStruct(e_t.shape, e_t.dtype),
        ),
        in_specs=[
            pl.BlockSpec((bx, d_feat), lambda i: (i, 0)),
            pl.BlockSpec((d_edge, be), lambda i: (0, i)),
        ],
        out_specs=(
            pl.BlockSpec((bx, d_feat), lambda i: (i, 0)),
            pl.BlockSpec((d_edge, be), lambda i: (0, i)),
        ),
    )(x, e_t)
    return (x_out, e_out_t.T)


# speedup vs baseline: 14.3065x; 1.3894x over previous
import jax
import jax.numpy as jnp
from jax.experimental import pallas as pl


def _copy_body(x_ref, e_ref, x_out_ref, e_out_ref):
    x_out_ref[...] = x_ref[...]
    e_out_ref[...] = e_ref[...]


def kernel(x, edge_index, edge_attr):
    del edge_index
    n_nodes, d_feat = x.shape
    n_edges, d_edge = edge_attr.shape
    e_t = edge_attr.T

    grid = 10
    bx = n_nodes // grid
    be = n_edges // grid

    x_out, e_out_t = pl.pallas_call(
        _copy_body,
        grid=(grid,),
        out_shape=(
            jax.ShapeDtypeStruct(x.shape, x.dtype),
            jax.ShapeDtypeStruct(e_t.shape, e_t.dtype),
        ),
        in_specs=[
            pl.BlockSpec((bx, d_feat), lambda i: (i, 0)),
            pl.BlockSpec((d_edge, be), lambda i: (0, i)),
        ],
        out_specs=(
            pl.BlockSpec((bx, d_feat), lambda i: (i, 0)),
            pl.BlockSpec((d_edge, be), lambda i: (0, i)),
        ),
    )(x, e_t)
    return (x_out, e_out_t.T)


# speedup vs baseline: 15.4377x; 1.0791x over previous
import jax
import jax.numpy as jnp
from jax.experimental import pallas as pl


def _copy_body(x_ref, e_ref, x_out_ref, e_out_ref):
    x_out_ref[...] = x_ref[...]
    e_out_ref[...] = e_ref[...]


def kernel(x, edge_index, edge_attr):
    del edge_index
    n_nodes, d_feat = x.shape
    n_edges, d_edge = edge_attr.shape
    e_t = edge_attr.T

    grid = 5
    bx = n_nodes // grid
    be = n_edges // grid

    x_out, e_out_t = pl.pallas_call(
        _copy_body,
        grid=(grid,),
        out_shape=(
            jax.ShapeDtypeStruct(x.shape, x.dtype),
            jax.ShapeDtypeStruct(e_t.shape, e_t.dtype),
        ),
        in_specs=[
            pl.BlockSpec((bx, d_feat), lambda i: (i, 0)),
            pl.BlockSpec((d_edge, be), lambda i: (0, i)),
        ],
        out_specs=(
            pl.BlockSpec((bx, d_feat), lambda i: (i, 0)),
            pl.BlockSpec((d_edge, be), lambda i: (0, i)),
        ),
    )(x, e_t)
    return (x_out, e_out_t.T)


# speedup vs baseline: 17.2037x; 1.1144x over previous
import jax
import jax.numpy as jnp
from jax.experimental import pallas as pl


def _copy_body(x_ref, e_ref, x_out_ref, e_out_ref):
    x_out_ref[...] = x_ref[...]
    e_out_ref[...] = e_ref[...]


def kernel(x, edge_index, edge_attr):
    del edge_index
    n_nodes, d_feat = x.shape
    n_edges, d_edge = edge_attr.shape
    e_t = edge_attr.T

    grid = 2
    bx = n_nodes // grid
    be = n_edges // grid

    x_out, e_out_t = pl.pallas_call(
        _copy_body,
        grid=(grid,),
        out_shape=(
            jax.ShapeDtypeStruct(x.shape, x.dtype),
            jax.ShapeDtypeStruct(e_t.shape, e_t.dtype),
        ),
        in_specs=[
            pl.BlockSpec((bx, d_feat), lambda i: (i, 0)),
            pl.BlockSpec((d_edge, be), lambda i: (0, i)),
        ],
        out_specs=(
            pl.BlockSpec((bx, d_feat), lambda i: (i, 0)),
            pl.BlockSpec((d_edge, be), lambda i: (0, i)),
        ),
    )(x, e_t)
    return (x_out, e_out_t.T)
